# baseline (speedup 1.0000x reference)
import functools

import jax
import jax.numpy as jnp
from jax import lax
from jax.experimental import pallas as pl
from jax.experimental.pallas import tpu as pltpu
from jax.experimental.pallas import tpu_sc as plsc


def _make_gather(n_rows_pad, d, n_chunks, chunk):
    mesh = plsc.VectorSubcoreMesh(core_axis_name="c", subcore_axis_name="s")
    nc = 2

    @functools.partial(
        pl.kernel,
        mesh=mesh,
        out_type=jax.ShapeDtypeStruct((n_rows_pad, d), jnp.float32),
        scratch_types=[
            pltpu.VMEM((n_chunks, chunk), jnp.int32),
            pltpu.VMEM((chunk, d), jnp.float32),
            pltpu.VMEM((chunk, d), jnp.float32),
            pltpu.SemaphoreType.DMA,
            pltpu.SemaphoreType.DMA,
            pltpu.SemaphoreType.DMA,
            pltpu.SemaphoreType.DMA,
        ],
    )
    def gather_k(table_hbm, idx_hbm, out_hbm, idx_v, rows0, rows1, g0, g1, w0, w1):
        wid = lax.axis_index("s") * nc + lax.axis_index("c")
        base = wid * (n_chunks * chunk)
        bufs = (rows0, rows1)
        gsems = (g0, g1)
        wsems = (w0, w1)

        pltpu.sync_copy(idx_hbm.at[wid], idx_v)

        gathers = [None] * n_chunks
        writes = [None] * n_chunks
        gathers[0] = pltpu.async_copy(table_hbm.at[idx_v.at[0]], bufs[0], gsems[0])
        for c in range(n_chunks):
            b = c % 2
            gathers[c].wait()
            if c >= 1:
                writes[c - 1].wait()
            if c + 1 < n_chunks:
                gathers[c + 1] = pltpu.async_copy(
                    table_hbm.at[idx_v.at[c + 1]], bufs[1 - b], gsems[1 - b]
                )
            writes[c] = pltpu.async_copy(
                bufs[b], out_hbm.at[pl.ds(base + c * chunk, chunk)], wsems[b]
            )
        writes[n_chunks - 1].wait()

    return gather_k


def kernel(sequence_tensor, positions):
    bt, seq, d = sequence_tensor.shape
    _, p = positions.shape
    n_rows = bt * p

    n_workers = 32
    chunk = 40
    per_w = -(-n_rows // (n_workers * chunk)) * chunk
    n_chunks = per_w // chunk
    n_rows_pad = per_w * n_workers

    table = sequence_tensor.reshape(bt * seq, d)
    idx = (positions.astype(jnp.int32) + jnp.arange(bt, dtype=jnp.int32)[:, None] * seq)
    idx = idx.reshape(-1)
    idx = jnp.pad(idx, (0, n_rows_pad - n_rows))
    idx = idx.reshape(n_workers, n_chunks, chunk)

    out = _make_gather(n_rows_pad, d, n_chunks, chunk)(table, idx)
    return out[:n_rows].reshape(bt, p, d)


# speedup vs baseline: 1.2670x; 1.2670x over previous
import functools

import jax
import jax.numpy as jnp
from jax import lax
from jax.experimental import pallas as pl
from jax.experimental.pallas import tpu as pltpu
from jax.experimental.pallas import tpu_sc as plsc


def _make_gather(n_rows, d, n_chunks, chunk):
    mesh = plsc.VectorSubcoreMesh(core_axis_name="c", subcore_axis_name="s")
    nc = 2
    per_w = n_chunks * chunk

    @functools.partial(
        pl.kernel,
        mesh=mesh,
        out_type=jax.ShapeDtypeStruct((n_rows, d), jnp.float32),
        scratch_types=[
            pltpu.VMEM((n_chunks, chunk), jnp.int32),
            pltpu.VMEM((chunk, d), jnp.float32),
            pltpu.VMEM((chunk, d), jnp.float32),
            pltpu.SemaphoreType.DMA,
            pltpu.SemaphoreType.DMA,
            pltpu.SemaphoreType.DMA,
            pltpu.SemaphoreType.DMA,
        ],
    )
    def gather_k(table_hbm, idx_hbm, out_hbm, idx_v, rows0, rows1, g0, g1, w0, w1):
        wid = lax.axis_index("s") * nc + lax.axis_index("c")
        base = (wid * (n_rows - per_w) // 31) // 8 * 8
        bufs = (rows0, rows1)
        gsems = (g0, g1)
        wsems = (w0, w1)

        pltpu.sync_copy(idx_hbm.at[wid], idx_v)

        gathers = [None] * n_chunks
        writes = [None] * n_chunks
        gathers[0] = pltpu.async_copy(table_hbm.at[idx_v.at[0]], bufs[0], gsems[0])
        for c in range(n_chunks):
            b = c % 2
            gathers[c].wait()
            if c >= 1:
                writes[c - 1].wait()
            if c + 1 < n_chunks:
                gathers[c + 1] = pltpu.async_copy(
                    table_hbm.at[idx_v.at[c + 1]], bufs[1 - b], gsems[1 - b]
                )
            writes[c] = pltpu.async_copy(
                bufs[b], out_hbm.at[pl.ds(base + c * chunk, chunk)], wsems[b]
            )
        writes[n_chunks - 1].wait()

    return gather_k


def kernel(sequence_tensor, positions):
    bt, seq, d = sequence_tensor.shape
    _, p = positions.shape
    n_rows = bt * p

    n_workers = 32
    n_chunks = 4
    chunk = 40
    per_w = n_chunks * chunk

    idx = (positions.astype(jnp.int32) + jnp.arange(bt, dtype=jnp.int32)[:, None] * seq)
    idx = idx.reshape(-1)
    starts = (jnp.arange(n_workers, dtype=jnp.int32) * (n_rows - per_w) // 31) // 8 * 8
    idx3d = idx[starts[:, None] + jnp.arange(per_w, dtype=jnp.int32)[None, :]]
    idx3d = idx3d.reshape(n_workers, n_chunks, chunk)

    table = sequence_tensor.reshape(bt * seq, d)
    out = _make_gather(n_rows, d, n_chunks, chunk)(table, idx3d)
    return out.reshape(bt, p, d)


# speedup vs baseline: 1.3799x; 1.0891x over previous
import functools

import jax
import jax.numpy as jnp
from jax import lax
from jax.experimental import pallas as pl
from jax.experimental.pallas import tpu as pltpu
from jax.experimental.pallas import tpu_sc as plsc


def _make_gather(bt, p, d, n_chunks, chunk, stride, blk):
    mesh = plsc.VectorSubcoreMesh(core_axis_name="c", subcore_axis_name="s")
    nc = 2
    per_w = n_chunks * chunk
    tail_start = p // 8 * 8
    tail = p - tail_start

    @functools.partial(
        pl.kernel,
        mesh=mesh,
        out_type=jax.ShapeDtypeStruct((bt, p, d), jnp.float32),
        scratch_types=[
            pltpu.VMEM((n_chunks + 1, chunk), jnp.int32),
            pltpu.VMEM((chunk, d), jnp.float32),
            pltpu.VMEM((chunk, d), jnp.float32),
            pltpu.VMEM((8, d), jnp.float32),
            pltpu.SemaphoreType.DMA,
            pltpu.SemaphoreType.DMA,
            pltpu.SemaphoreType.DMA,
            pltpu.SemaphoreType.DMA,
            pltpu.SemaphoreType.DMA,
        ],
    )
    def gather_k(table_hbm, idx_hbm, out_hbm, idx_v, rows0, rows1,
                 tail_v, g0, g1, w0, w1, tsem):
        wid = lax.axis_index("s") * nc + lax.axis_index("c")
        b = wid // 8
        start = (wid % 8) * stride
        bufs = (rows0, rows1)
        gsems = (g0, g1)
        wsems = (w0, w1)

        pltpu.sync_copy(idx_hbm.at[wid], idx_v)

        tail_gather = pltpu.async_copy(
            table_hbm.at[idx_v.at[n_chunks, pl.ds(0, 8)]], tail_v, tsem
        )
        gathers = [None] * n_chunks
        writes = [None] * n_chunks
        gathers[0] = pltpu.async_copy(
            table_hbm.at[idx_v.at[0]], bufs[0], gsems[0]
        )
        for c in range(n_chunks):
            bf = c % 2
            gathers[c].wait()
            if c >= 1:
                writes[c - 1].wait()
            if c + 1 < n_chunks:
                gathers[c + 1] = pltpu.async_copy(
                    table_hbm.at[idx_v.at[c + 1]],
                    bufs[1 - bf], gsems[1 - bf],
                )
            writes[c] = pltpu.async_copy(
                bufs[bf], out_hbm.at[b, pl.ds(start + c * chunk, chunk)],
                wsems[bf],
            )
        tail_gather.wait()
        tail_write = pltpu.async_copy(
            tail_v.at[pl.ds(0, tail)], out_hbm.at[b, pl.ds(tail_start, tail)], tsem
        )
        writes[n_chunks - 1].wait()
        tail_write.wait()

    return gather_k


def kernel(sequence_tensor, positions):
    bt, seq, d = sequence_tensor.shape
    _, p = positions.shape

    n_slots = 8
    n_chunks = 4
    chunk = 40
    per_w = n_chunks * chunk
    stride = 152
    tail_start = p // 8 * 8
    tail = p - tail_start

    idx = positions.astype(jnp.int32) + jnp.arange(bt, dtype=jnp.int32)[:, None] * seq
    wins = jnp.stack(
        [idx[:, s * stride:s * stride + per_w] for s in range(n_slots)], axis=1
    ).reshape(bt, n_slots, n_chunks, chunk)
    t = jnp.broadcast_to(
        jnp.tile(idx[:, tail_start:p], chunk // tail)[:, None, None, :],
        (bt, n_slots, 1, chunk),
    )
    blocks = jnp.concatenate([wins, t], axis=2)
    blocks = blocks.reshape(bt * n_slots, n_chunks + 1, chunk)

    table = sequence_tensor.reshape(bt * seq, d)
    return _make_gather(bt, p, d, n_chunks, chunk, stride, 0)(table, blocks)


# speedup vs baseline: 2.4653x; 1.7866x over previous
import functools

import jax
import jax.numpy as jnp
from jax import lax
from jax.experimental import pallas as pl
from jax.experimental.pallas import tpu as pltpu
from jax.experimental.pallas import tpu_sc as plsc


def _make_gather(bt, p, d, n_chunks, chunk_p, n_workers):
    mesh = plsc.VectorSubcoreMesh(core_axis_name="c", subcore_axis_name="s")
    nc = 2
    rows_per_chunk = chunk_p * bt
    per_w = n_chunks * chunk_p

    @functools.partial(
        pl.kernel,
        mesh=mesh,
        out_type=jax.ShapeDtypeStruct((p, bt, d), jnp.float32),
        scratch_types=[
            pltpu.VMEM((n_chunks, rows_per_chunk), jnp.int32),
            pltpu.VMEM((chunk_p, bt, d), jnp.float32),
            pltpu.VMEM((chunk_p, bt, d), jnp.float32),
            pltpu.SemaphoreType.DMA,
            pltpu.SemaphoreType.DMA,
            pltpu.SemaphoreType.DMA,
            pltpu.SemaphoreType.DMA,
        ],
    )
    def gather_k(table_hbm, idx_hbm, out_hbm, idx_v, rows0, rows1, g0, g1, w0, w1):
        wid = lax.axis_index("s") * nc + lax.axis_index("c")
        start = jnp.minimum(wid * (p - per_w) // (n_workers - 1), p - per_w)
        bufs = (rows0, rows1)
        gsems = (g0, g1)
        wsems = (w0, w1)

        pltpu.sync_copy(idx_hbm.at[wid], idx_v)

        gathers = [None] * n_chunks
        writes = [None] * n_chunks
        gathers[0] = pltpu.async_copy(
            table_hbm.at[idx_v.at[0]], bufs[0].reshape(rows_per_chunk, d), gsems[0]
        )
        for c in range(n_chunks):
            bf = c % 2
            gathers[c].wait()
            if c >= 1:
                writes[c - 1].wait()
            if c + 1 < n_chunks:
                gathers[c + 1] = pltpu.async_copy(
                    table_hbm.at[idx_v.at[c + 1]],
                    bufs[1 - bf].reshape(rows_per_chunk, d), gsems[1 - bf],
                )
            writes[c] = pltpu.async_copy(
                bufs[bf],
                out_hbm.at[pl.ds(start + c * chunk_p, chunk_p)],
                wsems[bf],
            )
        writes[n_chunks - 1].wait()

    return gather_k


def kernel(sequence_tensor, positions):
    bt, seq, d = sequence_tensor.shape
    _, p = positions.shape

    n_workers = 32
    n_chunks = 4
    chunk_p = 10
    per_w = n_chunks * chunk_p

    starts = [
        min(w * (p - per_w) // (n_workers - 1), p - per_w)
        for w in range(n_workers)
    ]

    idx = positions.astype(jnp.int32) + jnp.arange(bt, dtype=jnp.int32)[:, None] * seq
    idx_t = idx.T.reshape(-1)
    blocks = jnp.stack(
        [idx_t[s * bt:(s + per_w) * bt] for s in starts]
    ).reshape(n_workers, n_chunks, chunk_p * bt)

    table = sequence_tensor.reshape(bt * seq, d)
    out_t = _make_gather(bt, p, d, n_chunks, chunk_p, n_workers)(table, blocks)
    return jnp.transpose(out_t, (1, 0, 2))


# speedup vs baseline: 2.6192x; 1.0624x over previous
import functools

import jax
import jax.numpy as jnp
from jax import lax
from jax.experimental import pallas as pl
from jax.experimental.pallas import tpu as pltpu
from jax.experimental.pallas import tpu_sc as plsc


def _make_gather(bt, p, d, n_chunks, chunk_p, n_workers):
    mesh = plsc.VectorSubcoreMesh(core_axis_name="c", subcore_axis_name="s")
    nc = 2
    rows_per_chunk = chunk_p * bt
    per_w = n_chunks * chunk_p

    @functools.partial(
        pl.kernel,
        mesh=mesh,
        out_type=jax.ShapeDtypeStruct((p, bt, d), jnp.float32),
        scratch_types=[
            pltpu.VMEM((n_chunks, rows_per_chunk), jnp.int32),
            pltpu.VMEM((chunk_p, bt, d), jnp.float32),
            pltpu.VMEM((chunk_p, bt, d), jnp.float32),
            pltpu.SemaphoreType.DMA,
            pltpu.SemaphoreType.DMA,
            pltpu.SemaphoreType.DMA,
            pltpu.SemaphoreType.DMA,
        ],
    )
    def gather_k(table_hbm, idx_hbm, out_hbm, idx_v, rows0, rows1, g0, g1, w0, w1):
        wid = lax.axis_index("s") * nc + lax.axis_index("c")
        start = jnp.minimum(wid * per_w, p - per_w)
        bufs = (rows0, rows1)
        gsems = (g0, g1)
        wsems = (w0, w1)

        pltpu.sync_copy(idx_hbm.at[wid], idx_v)

        gathers = [None] * n_chunks
        writes = [None] * n_chunks
        gathers[0] = pltpu.async_copy(
            table_hbm.at[idx_v.at[0]], bufs[0].reshape(rows_per_chunk, d), gsems[0]
        )
        for c in range(n_chunks):
            bf = c % 2
            gathers[c].wait()
            if c >= 1:
                writes[c - 1].wait()
            if c + 1 < n_chunks:
                gathers[c + 1] = pltpu.async_copy(
                    table_hbm.at[idx_v.at[c + 1]],
                    bufs[1 - bf].reshape(rows_per_chunk, d), gsems[1 - bf],
                )
            writes[c] = pltpu.async_copy(
                bufs[bf],
                out_hbm.at[pl.ds(start + c * chunk_p, chunk_p)],
                wsems[bf],
            )
        writes[n_chunks - 1].wait()

    return gather_k


def kernel(sequence_tensor, positions):
    bt, seq, d = sequence_tensor.shape
    _, p = positions.shape

    n_workers = 32
    n_chunks = 4
    chunk_p = 10
    per_w = n_chunks * chunk_p

    n_full = (p - per_w) // per_w + 1
    idx = positions.astype(jnp.int32) + jnp.arange(bt, dtype=jnp.int32)[:, None] * seq
    idx_t = idx.T.reshape(-1)
    full = idx_t[:n_full * per_w * bt].reshape(n_full, per_w * bt)
    last = jnp.broadcast_to(
        idx_t[(p - per_w) * bt:][None, :], (n_workers - n_full, per_w * bt)
    )
    blocks = jnp.concatenate([full, last]).reshape(
        n_workers, n_chunks, chunk_p * bt
    )

    table = sequence_tensor.reshape(bt * seq, d)
    out_t = _make_gather(bt, p, d, n_chunks, chunk_p, n_workers)(table, blocks)
    return jnp.transpose(out_t, (1, 0, 2))


# speedup vs baseline: 2.7955x; 1.0673x over previous
import functools

import jax
import jax.numpy as jnp
from jax import lax
from jax.experimental import pallas as pl
from jax.experimental.pallas import tpu as pltpu
from jax.experimental.pallas import tpu_sc as plsc


def _make_gather(bt, p, d, n_chunks, chunk_p, n_workers):
    mesh = plsc.VectorSubcoreMesh(core_axis_name="c", subcore_axis_name="s")
    nc = 2
    rows_per_chunk = chunk_p * bt
    per_w = n_chunks * chunk_p

    nbuf = 4

    @functools.partial(
        pl.kernel,
        mesh=mesh,
        out_type=jax.ShapeDtypeStruct((p, bt, d), jnp.float32),
        scratch_types=[
            pltpu.VMEM((n_chunks, rows_per_chunk), jnp.int32),
            *[pltpu.VMEM((chunk_p, bt, d), jnp.float32)] * nbuf,
            *[pltpu.SemaphoreType.DMA] * (2 * nbuf),
        ],
    )
    def gather_k(table_hbm, idx_hbm, out_hbm, idx_v, *bufs_sems):
        bufs = bufs_sems[:nbuf]
        gsems = bufs_sems[nbuf:2 * nbuf]
        wsems = bufs_sems[2 * nbuf:]
        wid = lax.axis_index("s") * nc + lax.axis_index("c")
        start = jnp.minimum(wid * per_w, p - per_w)

        pltpu.sync_copy(idx_hbm.at[wid], idx_v)

        def gather(c):
            b = c % nbuf
            return pltpu.async_copy(
                table_hbm.at[idx_v.at[c]],
                bufs[b].reshape(rows_per_chunk, d), gsems[b],
            )

        gathers = [None] * n_chunks
        writes = [None] * n_chunks
        for c in range(min(nbuf - 1, n_chunks)):
            gathers[c] = gather(c)
        for c in range(n_chunks):
            b = c % nbuf
            gathers[c].wait()
            writes[c] = pltpu.async_copy(
                bufs[b], out_hbm.at[pl.ds(start + c * chunk_p, chunk_p)], wsems[b]
            )
            if c + nbuf - 1 < n_chunks:
                if c >= 1:
                    writes[c - 1].wait()
                gathers[c + nbuf - 1] = gather(c + nbuf - 1)
        for c in range(max(n_chunks - nbuf, 0), n_chunks):
            writes[c].wait()

    return gather_k


def kernel(sequence_tensor, positions):
    bt, seq, d = sequence_tensor.shape
    _, p = positions.shape

    n_workers = 32
    n_chunks = 8
    chunk_p = 5
    per_w = n_chunks * chunk_p

    n_full = (p - per_w) // per_w + 1
    idx = positions.astype(jnp.int32) + jnp.arange(bt, dtype=jnp.int32)[:, None] * seq
    idx_t = idx.T.reshape(-1)
    full = idx_t[:n_full * per_w * bt].reshape(n_full, per_w * bt)
    last = jnp.broadcast_to(
        idx_t[(p - per_w) * bt:][None, :], (n_workers - n_full, per_w * bt)
    )
    blocks = jnp.concatenate([full, last]).reshape(
        n_workers, n_chunks, chunk_p * bt
    )

    table = sequence_tensor.reshape(bt * seq, d)
    out_t = _make_gather(bt, p, d, n_chunks, chunk_p, n_workers)(table, blocks)
    return jnp.transpose(out_t, (1, 0, 2))


# speedup vs baseline: 2.7958x; 1.0001x over previous
import functools

import jax
import jax.numpy as jnp
from jax import lax
from jax.experimental import pallas as pl
from jax.experimental.pallas import tpu as pltpu
from jax.experimental.pallas import tpu_sc as plsc


def _make_gather(bt, p, d, n_chunks, chunk_p, n_workers):
    mesh = plsc.VectorSubcoreMesh(core_axis_name="c", subcore_axis_name="s")
    nc = 2
    rows_per_chunk = chunk_p * bt
    per_w = n_chunks * chunk_p

    nbuf = 4

    @functools.partial(
        pl.kernel,
        mesh=mesh,
        out_type=jax.ShapeDtypeStruct((p, bt, d), jnp.float32),
        scratch_types=[
            pltpu.VMEM((n_chunks, rows_per_chunk), jnp.int32),
            *[pltpu.VMEM((chunk_p, bt, d), jnp.float32)] * nbuf,
            *[pltpu.SemaphoreType.DMA] * (2 * nbuf),
        ],
    )
    def gather_k(table_hbm, idx_hbm, out_hbm, idx_v, *bufs_sems):
        bufs = bufs_sems[:nbuf]
        gsems = bufs_sems[nbuf:2 * nbuf]
        wsems = bufs_sems[2 * nbuf:]
        wid = lax.axis_index("s") * nc + lax.axis_index("c")
        start = jnp.minimum(wid * per_w, p - per_w)

        pltpu.sync_copy(idx_hbm.at[wid], idx_v)

        def gather(c):
            b = c % nbuf
            return pltpu.async_copy(
                table_hbm.at[idx_v.at[c]],
                bufs[b].reshape(rows_per_chunk, d), gsems[b],
            )

        gathers = [None] * n_chunks
        writes = [None] * n_chunks
        for c in range(min(nbuf - 1, n_chunks)):
            gathers[c] = gather(c)
        for c in range(n_chunks):
            b = c % nbuf
            gathers[c].wait()
            writes[c] = pltpu.async_copy(
                bufs[b], out_hbm.at[pl.ds(start + c * chunk_p, chunk_p)], wsems[b]
            )
            if c + nbuf - 1 < n_chunks:
                if c >= 1:
                    writes[c - 1].wait()
                gathers[c + nbuf - 1] = gather(c + nbuf - 1)
        for c in range(max(n_chunks - nbuf, 0), n_chunks):
            writes[c].wait()

    return gather_k


def kernel(sequence_tensor, positions):
    bt, seq, d = sequence_tensor.shape
    _, p = positions.shape

    n_workers = 32
    n_chunks = 10
    chunk_p = 4
    per_w = n_chunks * chunk_p

    n_full = (p - per_w) // per_w + 1
    idx = positions.astype(jnp.int32) + jnp.arange(bt, dtype=jnp.int32)[:, None] * seq
    idx_t = idx.T.reshape(-1)
    full = idx_t[:n_full * per_w * bt].reshape(n_full, per_w * bt)
    last = jnp.broadcast_to(
        idx_t[(p - per_w) * bt:][None, :], (n_workers - n_full, per_w * bt)
    )
    blocks = jnp.concatenate([full, last]).reshape(
        n_workers, n_chunks, chunk_p * bt
    )

    table = sequence_tensor.reshape(bt * seq, d)
    out_t = _make_gather(bt, p, d, n_chunks, chunk_p, n_workers)(table, blocks)
    return jnp.transpose(out_t, (1, 0, 2))


# speedup vs baseline: 2.8010x; 1.0019x over previous
import functools

import jax
import jax.numpy as jnp
from jax import lax
from jax.experimental import pallas as pl
from jax.experimental.pallas import tpu as pltpu
from jax.experimental.pallas import tpu_sc as plsc


def _make_gather(bt, p, d, n_chunks, chunk_p, n_workers):
    mesh = plsc.VectorSubcoreMesh(core_axis_name="c", subcore_axis_name="s")
    nc = 2
    rows_per_chunk = chunk_p * bt
    per_w = n_chunks * chunk_p

    nbuf = 4

    @functools.partial(
        pl.kernel,
        mesh=mesh,
        out_type=jax.ShapeDtypeStruct((p, bt, d), jnp.float32),
        scratch_types=[
            pltpu.VMEM((n_chunks, rows_per_chunk), jnp.int32),
            *[pltpu.VMEM((chunk_p, bt, d), jnp.float32)] * nbuf,
            *[pltpu.SemaphoreType.DMA] * (2 * nbuf),
        ],
    )
    def gather_k(table_hbm, idx_hbm, out_hbm, idx_v, *bufs_sems):
        bufs = bufs_sems[:nbuf]
        gsems = bufs_sems[nbuf:2 * nbuf]
        wsems = bufs_sems[2 * nbuf:]
        wid = lax.axis_index("s") * nc + lax.axis_index("c")
        start = jnp.minimum(wid * per_w, p - per_w)

        pltpu.sync_copy(idx_hbm.at[wid], idx_v)

        def gather(c):
            b = c % nbuf
            return pltpu.async_copy(
                table_hbm.at[idx_v.at[c]],
                bufs[b].reshape(rows_per_chunk, d), gsems[b],
            )

        gathers = [None] * n_chunks
        writes = [None] * n_chunks
        for c in range(min(nbuf - 1, n_chunks)):
            gathers[c] = gather(c)
        for c in range(n_chunks):
            b = c % nbuf
            gathers[c].wait()
            writes[c] = pltpu.async_copy(
                bufs[b], out_hbm.at[pl.ds(start + c * chunk_p, chunk_p)], wsems[b]
            )
            if c + nbuf - 1 < n_chunks:
                if c >= 1:
                    writes[c - 1].wait()
                gathers[c + nbuf - 1] = gather(c + nbuf - 1)
        for c in range(max(n_chunks - nbuf, 0), n_chunks):
            writes[c].wait()

    return gather_k


def kernel(sequence_tensor, positions):
    bt, seq, d = sequence_tensor.shape
    _, p = positions.shape

    n_workers = 32
    n_chunks = 8
    chunk_p = 5
    per_w = n_chunks * chunk_p

    n_full = (p - per_w) // per_w + 1
    idx = positions.astype(jnp.int32) + jnp.arange(bt, dtype=jnp.int32)[:, None] * seq
    idx_t = idx.T.reshape(-1)
    full = idx_t[:n_full * per_w * bt].reshape(n_full, per_w * bt)
    last = jnp.broadcast_to(
        idx_t[(p - per_w) * bt:][None, :], (n_workers - n_full, per_w * bt)
    )
    blocks = jnp.concatenate([full, last]).reshape(
        n_workers, n_chunks, chunk_p * bt
    )

    table = sequence_tensor.reshape(bt * seq, d)
    out_t = _make_gather(bt, p, d, n_chunks, chunk_p, n_workers)(table, blocks)
    return jnp.transpose(out_t, (1, 0, 2))
